# baseline (speedup 1.0000x reference)
import functools

import jax
import jax.numpy as jnp
from jax import lax
from jax.experimental import pallas as pl
from jax.experimental.pallas import tpu as pltpu
from jax.experimental.pallas import tpu_sc as plsc

N = 10000
E = 320000
D = 128
C = 64
NC, NS, LANES = 2, 16, 16
NW = NC * NS
K = 128
CPW = 80
EPW = K * CPW
EPAD = EPW * NW
NCH_ALL = EPAD // K
NPAD = 10240
NPT = NPAD // NS
MMB = 400

_SC_PARAMS = pltpu.CompilerParams(needs_layout_passes=False)


def _mesh():
    return plsc.VectorSubcoreMesh(core_axis_name="c", subcore_axis_name="s")


def _splat(v):
    return jnp.zeros((LANES,), jnp.int32) + v


@functools.partial(
    pl.kernel,
    out_type=jax.ShapeDtypeStruct((NC, NPAD), jnp.float32),
    mesh=_mesh(),
    compiler_params=_SC_PARAMS,
    scratch_types=[
        pltpu.VMEM((NPAD,), jnp.float32),
        pltpu.VMEM((CPW, K), jnp.int32),
        pltpu.VMEM((CPW, K), jnp.float32),
        pltpu.VMEM((NS, NPT), jnp.float32),
        pltpu.VMEM_SHARED((NS, NPAD), jnp.float32),
    ],
)
def _deg_kernel(dpack_hbm, wpack_hbm, degp, acc, dbig, wbig, rbuf, shared):
    cid = lax.axis_index("c")
    sid = lax.axis_index("s")
    wid = cid * NS + sid

    def zero_body(i, c):
        acc[pl.ds(i * LANES, LANES)] = jnp.zeros((LANES,), jnp.float32)
        return c

    lax.fori_loop(0, NPAD // LANES, zero_body, 0)
    slab = pl.ds(pl.multiple_of(wid * CPW, 8), CPW)
    pltpu.sync_copy(dpack_hbm.at[slab], dbig)
    pltpu.sync_copy(wpack_hbm.at[slab], wbig)

    def chunk_body(c, carry):
        for i in range(K // LANES):
            idx = dbig[c, pl.ds(i * LANES, LANES)]
            val = wbig[c, pl.ds(i * LANES, LANES)]
            plsc.addupdate_scatter(acc, [idx], val)
        return carry

    lax.fori_loop(0, CPW, chunk_body, 0)

    pltpu.sync_copy(acc, shared.at[sid])
    plsc.subcore_barrier()
    col0 = pl.multiple_of(sid * NPT, 8)
    for r in range(NS):
        pltpu.sync_copy(shared.at[r, pl.ds(col0, NPT)], rbuf.at[r])

    def red_body(i, carry):
        s = rbuf[0, pl.ds(i * LANES, LANES)]
        for r in range(1, NS):
            s = s + rbuf[r, pl.ds(i * LANES, LANES)]
        acc[pl.ds(i * LANES, LANES)] = s
        return carry

    lax.fori_loop(0, NPT // LANES, red_body, 0)
    pltpu.sync_copy(acc.at[pl.ds(0, NPT)], degp.at[cid, pl.ds(col0, NPT)])


@functools.partial(
    pl.kernel,
    out_type=jax.ShapeDtypeStruct((NC, NPAD, D), jnp.float32),
    mesh=_mesh(),
    compiler_params=_SC_PARAMS,
    scratch_types=[
        [pltpu.VMEM((K,), jnp.int32) for _ in range(2)],
        [pltpu.VMEM((K,), jnp.int32) for _ in range(2)],
        pltpu.VMEM((CPW, K), jnp.float32),
        [pltpu.VMEM((K, D), jnp.float32) for _ in range(2)],
        pltpu.VMEM((32, D), jnp.float32),
        pltpu.VMEM_SHARED((NPAD, D), jnp.float32),
        [pltpu.SemaphoreType.DMA for _ in range(2)],
    ],
)
def _prop_kernel(h_hbm, spack_hbm, dpack_hbm, wpack_hbm, part,
                 sidx, didx, wbig, rows, zbuf, acc, gsem):
    cid = lax.axis_index("c")
    sid = lax.axis_index("s")
    wid = cid * NS + sid

    def zfill(i, c):
        for j in range(D // LANES):
            zbuf[i, pl.ds(j * LANES, LANES)] = jnp.zeros((LANES,), jnp.float32)
        return c

    lax.fori_loop(0, 32, zfill, 0)
    row0 = pl.multiple_of(sid * NPT, 8)

    def zcopy(k, carry):
        pltpu.sync_copy(zbuf, acc.at[pl.ds(pl.multiple_of(row0 + k * 32, 8),
                                           32)])
        return carry

    lax.fori_loop(0, NPT // 32, zcopy, 0)
    plsc.subcore_barrier()

    slab = pl.ds(pl.multiple_of(wid * CPW, 8), CPW)
    pltpu.sync_copy(wpack_hbm.at[slab], wbig)
    ebase = wid * EPW

    def gather_start(c, b):
        off = pl.multiple_of(ebase + c * K, 8)
        pltpu.sync_copy(spack_hbm.at[pl.ds(off, K)], sidx[b])
        pltpu.sync_copy(dpack_hbm.at[pl.ds(off, K)], didx[b])
        pltpu.async_copy(h_hbm.at[sidx[b]], rows[b], gsem[b])

    def gather_wait(c, b):
        pltpu.make_async_copy(h_hbm.at[sidx[b]], rows[b], gsem[b]).wait()

    def process(c, b):
        gather_wait(c, b)
        c16 = _splat(c)

        def sbody(e, cc):
            s = plsc.load_gather(wbig, [c16, _splat(e)])
            for j in range(D // LANES):
                rows[b][e, pl.ds(j * LANES, LANES)] = (
                    rows[b][e, pl.ds(j * LANES, LANES)] * s)
            return cc

        lax.fori_loop(0, K, sbody, 0)
        pltpu.sync_copy(rows[b], acc.at[didx[b]], add=True)

    gather_start(0, 0)

    def loop_body(i, carry):
        c = i * 2
        gather_start(c + 1, 1)
        process(c, 0)

        @pl.when(c + 2 < CPW)
        def _():
            gather_start(c + 2, 0)

        process(c + 1, 1)
        return carry

    lax.fori_loop(0, CPW // 2, loop_body, 0)
    plsc.subcore_barrier()

    def wb_body(k, carry):
        r = pl.multiple_of(row0 + k * 64, 8)
        pltpu.sync_copy(acc.at[pl.ds(r, 64)], part.at[cid, pl.ds(r, 64)])
        return carry

    lax.fori_loop(0, NPT // 64, wb_body, 0)


def _prep_body(degp_ref, dinv_ref):
    deg = degp_ref[0] + degp_ref[1] + 1.0
    dinv_ref[...] = lax.rsqrt(deg)


_prep = pl.pallas_call(
    _prep_body,
    out_shape=jax.ShapeDtypeStruct((NPAD // 128, 128), jnp.float32),
)


def _mm_body(a_ref, w_ref, dv_ref, o_ref):
    o_ref[...] = dv_ref[...] * jnp.dot(a_ref[...], w_ref[...],
                                       preferred_element_type=jnp.float32,
                                       precision=lax.Precision.HIGHEST)


_mm = pl.pallas_call(
    _mm_body,
    grid=(N // MMB,),
    in_specs=[pl.BlockSpec((MMB, D), lambda i: (i, 0)),
              pl.BlockSpec((D, D), lambda i: (0, 0)),
              pl.BlockSpec((MMB, 1), lambda i: (i, 0))],
    out_specs=pl.BlockSpec((MMB, D), lambda i: (i, 0)),
    out_shape=jax.ShapeDtypeStruct((N, D), jnp.float32),
)


def _fuse_mm_body(p0_ref, p1_ref, h_ref, dv_ref, b_ref, w_ref, o_ref):
    a = dv_ref[...] * (p0_ref[0] + p1_ref[0] + h_ref[...]) + b_ref[...]
    a = jnp.maximum(a, 0.0)
    o_ref[...] = dv_ref[...] * jnp.dot(a, w_ref[...],
                                       preferred_element_type=jnp.float32,
                                       precision=lax.Precision.HIGHEST)


_fuse_mm = pl.pallas_call(
    _fuse_mm_body,
    grid=(N // MMB,),
    in_specs=[pl.BlockSpec((1, MMB, D), lambda i: (0, i, 0)),
              pl.BlockSpec((1, MMB, D), lambda i: (1, i, 0)),
              pl.BlockSpec((MMB, D), lambda i: (i, 0)),
              pl.BlockSpec((MMB, 1), lambda i: (i, 0)),
              pl.BlockSpec((1, D), lambda i: (0, 0)),
              pl.BlockSpec((D, D), lambda i: (0, 0))],
    out_specs=pl.BlockSpec((MMB, D), lambda i: (i, 0)),
    out_shape=jax.ShapeDtypeStruct((N, D), jnp.float32),
)


def _final_body(p0_ref, p1_ref, h_ref, dv_ref, b_ref, w_ref, bo_ref, o_ref):
    a = dv_ref[...] * (p0_ref[0] + p1_ref[0] + h_ref[...]) + b_ref[...]
    a = jnp.maximum(a, 0.0)
    logits = jnp.dot(a, w_ref[...],
                     preferred_element_type=jnp.float32,
                     precision=lax.Precision.HIGHEST) + bo_ref[...]
    col = lax.broadcasted_iota(jnp.int32, (MMB, 128), 1)
    lm = jnp.where(col < C, logits, jnp.float32(-1e30))
    m = jnp.max(lm, axis=1, keepdims=True)
    ex = jnp.where(col < C, jnp.exp(lm - m), 0.0)
    o_ref[...] = ex / jnp.sum(ex, axis=1, keepdims=True)


_final = pl.pallas_call(
    _final_body,
    grid=(N // MMB,),
    in_specs=[pl.BlockSpec((1, MMB, D), lambda i: (0, i, 0)),
              pl.BlockSpec((1, MMB, D), lambda i: (1, i, 0)),
              pl.BlockSpec((MMB, D), lambda i: (i, 0)),
              pl.BlockSpec((MMB, 1), lambda i: (i, 0)),
              pl.BlockSpec((1, D), lambda i: (0, 0)),
              pl.BlockSpec((D, 128), lambda i: (0, 0)),
              pl.BlockSpec((1, 128), lambda i: (0, 0))],
    out_specs=pl.BlockSpec((MMB, 128), lambda i: (i, 0)),
    out_shape=jax.ShapeDtypeStruct((N, 128), jnp.float32),
)


def kernel(x, edge_index, edge_weight, W1, b1, W2, b2, Wout, bout):
    src = edge_index[0].astype(jnp.int32)
    dst = edge_index[1].astype(jnp.int32)
    w = edge_weight.astype(jnp.float32)
    pad = EPAD - E
    spack = jnp.concatenate([src, jnp.zeros((pad,), jnp.int32)]).reshape(
        NCH_ALL, K)
    dpack = jnp.concatenate([dst, jnp.zeros((pad,), jnp.int32)]).reshape(
        NCH_ALL, K)
    wpack = jnp.concatenate([w, jnp.zeros((pad,), jnp.float32)]).reshape(
        NCH_ALL, K)
    degp = _deg_kernel(dpack, wpack)
    dinv = _prep(degp.reshape(NC, NPAD // 128, 128))
    dinv_col = dinv.reshape(NPAD)[:N, None]

    h1 = _mm(x, W1, dinv_col)
    part1 = _prop_kernel(h1, spack.reshape(EPAD), dpack.reshape(EPAD), wpack)
    h2 = _fuse_mm(part1, part1, h1, dinv_col, b1.reshape(1, D), W2)
    part2 = _prop_kernel(h2, spack.reshape(EPAD), dpack.reshape(EPAD), wpack)
    Wout_pad = jnp.pad(Wout, ((0, 0), (0, 128 - C)))
    bout_pad = jnp.pad(bout, (0, 128 - C)).reshape(1, 128)
    outp = _final(part2, part2, h2, dinv_col, b2.reshape(1, D),
                  Wout_pad, bout_pad)
    return outp[:, :C]
